# baseline (speedup 1.0000x reference)
import functools

import jax
import jax.numpy as jnp
from jax import lax
from jax.experimental import pallas as pl
from jax.experimental.pallas import tpu as pltpu
from jax.experimental.pallas import tpu_sc as plsc

IOU_THR = 0.4
BI = 512
BJ = 512
BIG = 1e30
NB = 64
NT = 16


def _bucket_body(pk_hbm, bu_hbm,
                 perm_hbm, bstart_hbm, hists_hbm,
                 pkv, buv, posv, histv, allh, basev, hist_s, cnt_s, sem):
    wid = lax.axis_index("s")
    m = bu_hbm.shape[0] // NT
    base = wid * m
    pltpu.sync_copy(pk_hbm.at[pl.ds(base, m)], pkv)
    pltpu.sync_copy(bu_hbm.at[pl.ds(base, m)], buv)

    lanes = lax.iota(jnp.int32, 16)

    for i in range(NB):
        hist_s[i] = jnp.int32(0)

    def hloop(c, carry):
        v = buv[pl.ds(c * 16, 16)]
        for l in range(16):
            b = v[l]
            hist_s[b] = hist_s[b] + 1
        return carry

    lax.fori_loop(0, m // 16, hloop, jnp.int32(0))

    for c in range(NB // 16):
        v = jnp.zeros((16,), jnp.int32)
        for l in range(16):
            v = jnp.where(lanes == l, hist_s[c * 16 + l], v)
        histv[pl.ds(c * 16, 16)] = v

    pltpu.sync_copy(histv, hists_hbm.at[wid])
    plsc.subcore_barrier()
    pltpu.sync_copy(hists_hbm, allh)

    carry = jnp.int32(0)
    for c in range(NB // 16):
        sl = pl.ds(c * 16, 16)
        tot = jnp.zeros((16,), jnp.int32)
        for t in range(NT):
            tot = tot + allh[t, sl]
        excl = jnp.zeros((16,), jnp.int32)
        for l in range(16):
            excl = jnp.where(lanes == l, carry, excl)
            carry = carry + tot[l]

        def mloop(t, mb):
            return mb + allh[t, sl]

        mybase = lax.fori_loop(0, wid, mloop, jnp.zeros((16,), jnp.int32))
        bvec = excl + mybase
        basev[sl] = bvec
        for l in range(16):
            cnt_s[c * 16 + l] = bvec[l]

    @pl.when(wid == 0)
    def _():
        pltpu.sync_copy(basev, bstart_hbm)

    def ploop(c, carry):
        v = buv[pl.ds(c * 16, 16)]
        pv = jnp.zeros((16,), jnp.int32)
        for l in range(16):
            b = v[l]
            p = cnt_s[b]
            cnt_s[b] = p + 1
            pv = jnp.where(lanes == l, p, pv)
        posv[pl.ds(c * 16, 16)] = pv
        return carry

    lax.fori_loop(0, m // 16, ploop, jnp.int32(0))

    pltpu.async_copy(pkv, perm_hbm.at[posv], sem).wait()


def _bucket_permute(pk, bu):
    npad = pk.shape[0]
    m = npad // NT
    mesh = plsc.VectorSubcoreMesh(core_axis_name="c", subcore_axis_name="s",
                                  num_cores=1)
    f = pl.kernel(
        _bucket_body,
        mesh=mesh,
        compiler_params=pltpu.CompilerParams(use_tc_tiling_on_sc=False),
        out_type=[
            jax.ShapeDtypeStruct((npad, 8), jnp.float32),
            jax.ShapeDtypeStruct((NB,), jnp.int32),
            jax.ShapeDtypeStruct((NT, NB), jnp.int32),
        ],
        scratch_types=[
            pltpu.VMEM((m, 8), jnp.float32),
            pltpu.VMEM((m,), jnp.int32),
            pltpu.VMEM((m,), jnp.int32),
            pltpu.VMEM((NB,), jnp.int32),
            pltpu.VMEM((NT, NB), jnp.int32),
            pltpu.VMEM((NB,), jnp.int32),
            pltpu.SMEM((NB,), jnp.int32),
            pltpu.SMEM((NB,), jnp.int32),
            pltpu.SemaphoreType.DMA,
        ],
    )
    return f(pk, bu)


def _nms_decay_body(c0_ref, c1_ref,
                    x1i_ref, y1i_ref, x2i_ref, y2i_ref, si_ref,
                    x1j_ref, y1j_ref, x2j_ref, y2j_ref, sj_ref,
                    out_ref):
    b = pl.program_id(0)
    lo = c0_ref[b]
    nch = c1_ref[b]

    x1i = x1i_ref[...]
    y1i = y1i_ref[...]
    x2i = x2i_ref[...]
    y2i = y2i_ref[...]
    si = si_ref[...]
    area_i = (x2i - x1i) * (y2i - y1i)

    def body(c, acc):
        sl = pl.ds(pl.multiple_of(lo + c * BJ, 128), BJ)
        x1j = x1j_ref[:, sl]
        y1j = y1j_ref[:, sl]
        x2j = x2j_ref[:, sl]
        y2j = y2j_ref[:, sl]
        sj = sj_ref[:, sl]
        area_j = (x2j - x1j) * (y2j - y1j)

        w = jnp.maximum(jnp.minimum(x2i, x2j) - jnp.maximum(x1i, x1j), 0.0)
        h = jnp.maximum(jnp.minimum(y2i, y2j) - jnp.maximum(y1i, y1j), 0.0)
        inter = w * h
        union = (area_i + area_j) - inter
        iou = inter / union
        cond = jnp.logical_and(iou > IOU_THR, sj > si)
        f = jnp.where(cond, 1.0 - iou, 1.0)
        return acc * f

    acc = jax.lax.fori_loop(0, nch, body,
                            jnp.ones((BI, BJ), jnp.float32))

    width = BJ
    while width > 1:
        width //= 2
        acc = acc[:, :width] * acc[:, width:2 * width]

    out_ref[...] = si * acc


@jax.jit
def kernel(boxes, scores):
    n = boxes.shape[0]
    npad = ((n + BI - 1) // BI) * BI
    pad = npad - n

    x1 = boxes[:, 0]
    y1 = boxes[:, 1]
    x2p = boxes[:, 2] + 1.0
    y2p = boxes[:, 3] + 1.0
    maxext = jnp.maximum(jnp.max(x2p - x1), jnp.max(y2p - y1))
    minx = jnp.min(x1)
    cell = jnp.maximum(maxext, (jnp.max(x1) - minx) / NB)

    fullc = lambda a, v: jnp.pad(a, (0, pad), constant_values=v)
    idxf = jnp.arange(npad, dtype=jnp.float32)
    x1f = fullc(x1, BIG)
    pk = jnp.stack([x1f, fullc(y1, BIG), fullc(x2p, BIG),
                    fullc(y2p, BIG), fullc(scores, -BIG), idxf,
                    jnp.zeros((npad,), jnp.float32),
                    jnp.zeros((npad,), jnp.float32)], axis=1)
    bu = jnp.clip((x1f - minx) / cell, 0.0, NB - 1).astype(jnp.int32)

    perm, bstart, _ = _bucket_permute(pk, bu)

    xs1 = jnp.pad(perm[:, 0], (0, BJ), constant_values=BIG)
    ys1 = jnp.pad(perm[:, 1], (0, BJ), constant_values=BIG)
    xs2 = jnp.pad(perm[:, 2], (0, BJ), constant_values=BIG)
    ys2 = jnp.pad(perm[:, 3], (0, BJ), constant_values=BIG)
    ss = jnp.pad(perm[:, 4], (0, BJ), constant_values=-BIG)
    order = perm[:, 5].astype(jnp.int32)

    nb = npad // BI
    bstart_ext = jnp.concatenate([bstart, jnp.array([npad], jnp.int32)])
    r0 = jnp.arange(nb, dtype=jnp.int32) * BI
    r1 = r0 + BI - 1
    bu0 = jnp.searchsorted(bstart, r0, side='right').astype(jnp.int32) - 1
    bu1 = jnp.searchsorted(bstart, r1, side='right').astype(jnp.int32) - 1
    lo_idx = bstart_ext[jnp.clip(bu0 - 1, 0, NB)]
    hi_idx = bstart_ext[jnp.clip(bu1 + 2, 0, NB)]
    lo_row = (lo_idx // 128 * 128).astype(jnp.int32)
    c0 = lo_row
    c1 = ((hi_idx - lo_row + BJ - 1) // BJ).astype(jnp.int32)

    col = lambda a: a[:npad].reshape(npad, 1)
    row = lambda a: a.reshape(1, npad + BJ)

    ispec = pl.BlockSpec((BI, 1), lambda i: (i, 0))
    jspec = pl.BlockSpec((1, npad + BJ), lambda i: (0, 0))
    sspec = pl.BlockSpec(memory_space=pltpu.SMEM)

    out = pl.pallas_call(
        _nms_decay_body,
        grid=(nb,),
        in_specs=[sspec, sspec,
                  ispec, ispec, ispec, ispec, ispec,
                  jspec, jspec, jspec, jspec, jspec],
        out_specs=pl.BlockSpec((BI, 1), lambda i: (i, 0)),
        out_shape=jax.ShapeDtypeStruct((npad, 1), jnp.float32),
    )(c0, c1,
      col(xs1), col(ys1), col(xs2), col(ys2), col(ss),
      row(xs1), row(ys1), row(xs2), row(ys2), row(ss))

    return jnp.zeros((npad,), jnp.float32).at[order].set(out[:, 0])[:n]
